# Initial kernel scaffold; baseline (speedup 1.0000x reference)
import jax
import jax.numpy as jnp
from jax.experimental import pallas as pl


def kernel(x, edge_index, edge_attr, W1, b1, W2, b2):
    raise NotImplementedError("write your pallas kernel here")


# speedup vs baseline: 20.2278x; 20.2278x over previous
import functools

import jax
import jax.numpy as jnp
from jax import lax
from jax.experimental import pallas as pl
from jax.experimental.pallas import tpu as pltpu
from jax.experimental.pallas import tpu_sc as plsc

N_NODES = 10000
N_EDGES = 160000
D_FEAT = 256
HIDDEN = 16
N_CLASSES = 2

NC = 2
NS = 16
NW = NC * NS
CH = 128
NCH = 40
EPW = CH * NCH
EP = NW * EPW
NPAD = 10240
ZR = NPAD // NS


def _sc_pass(do_gather):
    mesh = plsc.VectorSubcoreMesh(core_axis_name="c", subcore_axis_name="s")
    scratch = [
        pltpu.VMEM((NCH, CH), jnp.int32),
        pltpu.VMEM((NCH, CH), jnp.int32),
        pltpu.VMEM((NCH, CH), jnp.float32),
        pltpu.VMEM((EPW, HIDDEN), jnp.float32),
        pltpu.VMEM((ZR, HIDDEN), jnp.float32),
        pltpu.VMEM_SHARED((NPAD, HIDDEN), jnp.float32),
        pltpu.SemaphoreType.DMA,
    ]

    def body(row_hbm, col_hbm, ew_hbm, table_hbm, out_hbm,
             row_v, col_v, ew_v, msg_v, zbuf, accum, sem):
        c = lax.axis_index("c")
        s = lax.axis_index("s")
        wid = c * NS + s
        base = wid * NCH

        def zb(i, _):
            zbuf[i] = jnp.zeros((HIDDEN,), jnp.float32)
            return 0
        lax.fori_loop(0, ZR, zb, 0)
        pltpu.sync_copy(zbuf, accum.at[pl.ds(s * ZR, ZR)])

        pltpu.sync_copy(row_hbm.at[pl.ds(base, NCH)], row_v)
        pltpu.sync_copy(col_hbm.at[pl.ds(base, NCH)], col_v)
        pltpu.sync_copy(ew_hbm.at[pl.ds(base, NCH)], ew_v)

        if do_gather:
            def gchunk(j, _):
                pltpu.async_copy(table_hbm.at[row_v.at[j]],
                                 msg_v.at[pl.ds(j * CH, CH)], sem).wait()
                return 0
            lax.fori_loop(0, NCH, gchunk, 0)

            def sc_outer(j, _):
                def sc_mid(k16, _):
                    w = ew_v[j, pl.ds(k16 * 16, 16)]
                    for l in range(16):
                        k = j * CH + k16 * 16 + l
                        msg_v[k] = msg_v[k] * w[l]
                    return 0
                lax.fori_loop(0, CH // 16, sc_mid, 0)
                return 0
            lax.fori_loop(0, NCH, sc_outer, 0)
        else:
            def bc_outer(j, _):
                def bc_mid(k16, _):
                    w = ew_v[j, pl.ds(k16 * 16, 16)]
                    for l in range(16):
                        k = j * CH + k16 * 16 + l
                        msg_v[k] = jnp.ones((HIDDEN,), jnp.float32) * w[l]
                    return 0
                lax.fori_loop(0, CH // 16, bc_mid, 0)
                return 0
            lax.fori_loop(0, NCH, bc_outer, 0)

        plsc.subcore_barrier()

        def schunk(j, _):
            pltpu.sync_copy(msg_v.at[pl.ds(j * CH, CH)],
                            accum.at[col_v.at[j]], add=True)
            return 0
        lax.fori_loop(0, NCH, schunk, 0)

        plsc.subcore_barrier()

        pltpu.sync_copy(accum.at[pl.ds(s * ZR, ZR)],
                        out_hbm.at[c, pl.ds(s * ZR, ZR)])

    return pl.kernel(
        body,
        out_type=jax.ShapeDtypeStruct((NC, NPAD, HIDDEN), jnp.float32),
        mesh=mesh,
        scratch_types=scratch,
        compiler_params=pltpu.CompilerParams(use_tc_tiling_on_sc=False),
    )


def _tc1_body(x_ref, w1_ref, degp_ref, vs1_ref, dis_ref):
    deg = degp_ref[0] + degp_ref[1] + 1.0
    dis = 1.0 / jnp.sqrt(deg)
    xw = jnp.dot(x_ref[...], w1_ref[...], preferred_element_type=jnp.float32)
    vs1_ref[...] = dis * xw
    dis_ref[...] = dis


def _tc2_body(p1_ref, vs1_ref, dis_ref, b1_ref, vs2_ref):
    dis = dis_ref[...]
    agg = dis * (p1_ref[0] + p1_ref[1] + vs1_ref[...]) + b1_ref[...]
    vs2_ref[...] = dis * jnp.maximum(agg, 0.0)


def _tc3_body(p2_ref, vs2_ref, dis_ref, w2_ref, b2_ref, out_ref):
    t = dis_ref[...] * (p2_ref[0] + p2_ref[1] + vs2_ref[...])
    logits = jnp.dot(t, w2_ref[...],
                     preferred_element_type=jnp.float32) + b2_ref[...]
    m = jnp.max(logits, axis=1, keepdims=True)
    lse = m + jnp.log(jnp.sum(jnp.exp(logits - m), axis=1, keepdims=True))
    out_ref[...] = logits - lse


def kernel(x, edge_index, edge_attr, W1, b1, W2, b2):
    row = edge_index[0].astype(jnp.int32)
    col = edge_index[1].astype(jnp.int32)
    pad_e = EP - N_EDGES
    row2d = jnp.pad(row, (0, pad_e)).reshape(NW * NCH, CH)
    col2d = jnp.pad(col, (0, pad_e)).reshape(NW * NCH, CH)
    ew2d = jnp.pad(edge_attr, (0, pad_e)).reshape(NW * NCH, CH)
    x_pad = jnp.pad(x, ((0, NPAD - N_NODES), (0, 0)))
    ones_tab = jnp.ones((NPAD, HIDDEN), jnp.float32)
    b1r = b1.reshape(1, HIDDEN)
    b2r = b2.reshape(1, N_CLASSES)

    degp = _sc_pass(False)(row2d, col2d, ew2d, ones_tab)

    vs1, dis_b = pl.pallas_call(
        _tc1_body,
        out_shape=[jax.ShapeDtypeStruct((NPAD, HIDDEN), jnp.float32),
                   jax.ShapeDtypeStruct((NPAD, HIDDEN), jnp.float32)],
    )(x_pad, W1, degp)

    p1 = _sc_pass(True)(row2d, col2d, ew2d, vs1)

    vs2 = pl.pallas_call(
        _tc2_body,
        out_shape=jax.ShapeDtypeStruct((NPAD, HIDDEN), jnp.float32),
    )(p1, vs1, dis_b, b1r)

    p2 = _sc_pass(True)(row2d, col2d, ew2d, vs2)

    out = pl.pallas_call(
        _tc3_body,
        out_shape=jax.ShapeDtypeStruct((NPAD, N_CLASSES), jnp.float32),
    )(p2, vs2, dis_b, W2, b2r)

    return out[:N_NODES]


# speedup vs baseline: 25.9210x; 1.2815x over previous
import jax
import jax.numpy as jnp
from jax import lax
from jax.experimental import pallas as pl
from jax.experimental.pallas import tpu as pltpu
from jax.experimental.pallas import tpu_sc as plsc

N_NODES = 10000
N_EDGES = 160000
D_FEAT = 256
HIDDEN = 16
N_CLASSES = 2

NC = 2
NS = 16
NW = NC * NS
CH = 128
NCH = 40
EPW = CH * NCH
EP = NW * EPW
NPAD = 10240
ZR = NPAD // NS
PF = 6
SD = 6

_SC_PARAMS = pltpu.CompilerParams(use_tc_tiling_on_sc=False)


def _msg_pass():
    mesh = plsc.VectorSubcoreMesh(core_axis_name="c", subcore_axis_name="s")
    scratch = [
        pltpu.VMEM((NCH, CH), jnp.int32),
        pltpu.VMEM((NCH, CH), jnp.int32),
        pltpu.VMEM((NCH, CH), jnp.float32),
        pltpu.VMEM((EPW, HIDDEN), jnp.float32),
        pltpu.VMEM((ZR, HIDDEN), jnp.float32),
        pltpu.VMEM_SHARED((NPAD, HIDDEN), jnp.float32),
        pltpu.SemaphoreType.DMA,
        pltpu.SemaphoreType.DMA,
    ]

    def body(row_hbm, col_hbm, ew_hbm, table_hbm, out_hbm,
             row_v, col_v, ew_v, msg_v, zbuf, accum, gsem, ssem):
        c = lax.axis_index("c")
        s = lax.axis_index("s")
        base = (c * NS + s) * NCH

        def zb(i, _):
            zbuf[i] = jnp.zeros((HIDDEN,), jnp.float32)
            return 0
        lax.fori_loop(0, ZR, zb, 0)
        pltpu.sync_copy(zbuf, accum.at[pl.ds(s * ZR, ZR)])

        pltpu.sync_copy(row_hbm.at[pl.ds(base, NCH)], row_v)
        pltpu.sync_copy(col_hbm.at[pl.ds(base, NCH)], col_v)
        pltpu.sync_copy(ew_hbm.at[pl.ds(base, NCH)], ew_v)

        plsc.subcore_barrier()

        def fire_gather(j):
            pltpu.async_copy(table_hbm.at[row_v.at[j]],
                             msg_v.at[pl.ds(j * CH, CH)], gsem)

        def drain_gather(j):
            pltpu.make_async_copy(table_hbm.at[row_v.at[j]],
                                  msg_v.at[pl.ds(j * CH, CH)], gsem).wait()

        def fire_scatter(j):
            pltpu.async_copy(msg_v.at[pl.ds(j * CH, CH)],
                             accum.at[col_v.at[j]], ssem, add=True)

        def drain_scatter(j):
            pltpu.make_async_copy(msg_v.at[pl.ds(j * CH, CH)],
                                  accum.at[col_v.at[j]], ssem).wait()

        for j in range(PF):
            fire_gather(j)

        def step(j, _):
            @pl.when(j + PF < NCH)
            def _():
                fire_gather(j + PF)
            drain_gather(j)
            def sc_mid(k16, _):
                w = ew_v[j, pl.ds(k16 * 16, 16)]
                for l in range(16):
                    k = j * CH + k16 * 16 + l
                    msg_v[k] = msg_v[k] * w[l]
                return 0
            lax.fori_loop(0, CH // 16, sc_mid, 0)
            fire_scatter(j)
            @pl.when(j >= SD)
            def _():
                drain_scatter(j - SD)
            return 0
        lax.fori_loop(0, NCH, step, 0)

        def tail(j, _):
            drain_scatter(j)
            return 0
        lax.fori_loop(NCH - SD, NCH, tail, 0)

        plsc.subcore_barrier()

        pltpu.sync_copy(accum.at[pl.ds(s * ZR, ZR)],
                        out_hbm.at[c, pl.ds(s * ZR, ZR)])

    return pl.kernel(
        body,
        out_type=jax.ShapeDtypeStruct((NC, NPAD, HIDDEN), jnp.float32),
        mesh=mesh,
        scratch_types=scratch,
        compiler_params=_SC_PARAMS,
    )


def _deg_pass():
    mesh = plsc.VectorSubcoreMesh(core_axis_name="c", subcore_axis_name="s")
    scratch = [
        pltpu.VMEM((NCH, CH), jnp.int32),
        pltpu.VMEM((NCH, CH), jnp.float32),
        pltpu.VMEM((ZR,), jnp.float32),
        pltpu.VMEM((ZR, HIDDEN), jnp.float32),
        pltpu.VMEM_SHARED((NPAD,), jnp.float32),
        pltpu.SemaphoreType.DMA,
    ]

    def body(col_hbm, ew_hbm, out_hbm, col_v, ew_v, dbuf, obuf, accum, sem):
        c = lax.axis_index("c")
        s = lax.axis_index("s")
        base = (c * NS + s) * NCH

        def zb(i, _):
            dbuf[pl.ds(i * 16, 16)] = jnp.zeros((16,), jnp.float32)
            return 0
        lax.fori_loop(0, ZR // 16, zb, 0)
        pltpu.sync_copy(dbuf, accum.at[pl.ds(s * ZR, ZR)])

        pltpu.sync_copy(col_hbm.at[pl.ds(base, NCH)], col_v)
        pltpu.sync_copy(ew_hbm.at[pl.ds(base, NCH)], ew_v)

        plsc.subcore_barrier()

        def fire(j):
            pltpu.async_copy(ew_v.at[j], accum.at[col_v.at[j]], sem,
                             add=True)

        def drain(j):
            pltpu.make_async_copy(ew_v.at[j], accum.at[col_v.at[j]],
                                  sem).wait()

        for j in range(SD):
            fire(j)

        def step(j, _):
            @pl.when(j + SD < NCH)
            def _():
                fire(j + SD)
            drain(j)
            return 0
        lax.fori_loop(0, NCH, step, 0)

        plsc.subcore_barrier()

        pltpu.sync_copy(accum.at[pl.ds(s * ZR, ZR)], dbuf)

        def bc(i, _):
            v = dbuf[pl.ds(i * 16, 16)]
            for l in range(16):
                obuf[i * 16 + l] = jnp.ones((HIDDEN,), jnp.float32) * v[l]
            return 0
        lax.fori_loop(0, ZR // 16, bc, 0)
        pltpu.sync_copy(obuf, out_hbm.at[c, pl.ds(s * ZR, ZR)])

    return pl.kernel(
        body,
        out_type=jax.ShapeDtypeStruct((NC, NPAD, HIDDEN), jnp.float32),
        mesh=mesh,
        scratch_types=scratch,
        compiler_params=_SC_PARAMS,
    )


def _tc_mm_body(x_ref, w1_ref, xw_ref):
    xw_ref[...] = jnp.dot(x_ref[...], w1_ref[...],
                          preferred_element_type=jnp.float32)


def _tc1_body(xw_ref, degp_ref, vs1_ref, dis_ref):
    deg = degp_ref[0] + degp_ref[1] + 1.0
    dis = 1.0 / jnp.sqrt(deg)
    vs1_ref[...] = dis * xw_ref[...]
    dis_ref[...] = dis


def _tc2_body(p1_ref, vs1_ref, dis_ref, b1_ref, vs2_ref):
    dis = dis_ref[...]
    agg = dis * (p1_ref[0] + p1_ref[1] + vs1_ref[...]) + b1_ref[...]
    vs2_ref[...] = dis * jnp.maximum(agg, 0.0)


def _tc3_body(p2_ref, vs2_ref, dis_ref, w2_ref, b2_ref, out_ref):
    t = dis_ref[...] * (p2_ref[0] + p2_ref[1] + vs2_ref[...])
    logits = jnp.dot(t, w2_ref[...],
                     preferred_element_type=jnp.float32) + b2_ref[...]
    m = jnp.max(logits, axis=1, keepdims=True)
    lse = m + jnp.log(jnp.sum(jnp.exp(logits - m), axis=1, keepdims=True))
    out_ref[...] = logits - lse


def kernel(x, edge_index, edge_attr, W1, b1, W2, b2):
    row = edge_index[0].astype(jnp.int32)
    col = edge_index[1].astype(jnp.int32)
    pad_e = EP - N_EDGES
    row2d = jnp.pad(row, (0, pad_e)).reshape(NW * NCH, CH)
    col2d = jnp.pad(col, (0, pad_e)).reshape(NW * NCH, CH)
    ew2d = jnp.pad(edge_attr, (0, pad_e)).reshape(NW * NCH, CH)
    x_pad = jnp.pad(x, ((0, NPAD - N_NODES), (0, 0)))
    b1r = b1.reshape(1, HIDDEN)
    b2r = b2.reshape(1, N_CLASSES)

    degp = _deg_pass()(col2d, ew2d)
    xw = pl.pallas_call(
        _tc_mm_body,
        out_shape=jax.ShapeDtypeStruct((NPAD, HIDDEN), jnp.float32),
    )(x_pad, W1)

    vs1, dis_b = pl.pallas_call(
        _tc1_body,
        out_shape=[jax.ShapeDtypeStruct((NPAD, HIDDEN), jnp.float32),
                   jax.ShapeDtypeStruct((NPAD, HIDDEN), jnp.float32)],
    )(xw, degp)

    p1 = _msg_pass()(row2d, col2d, ew2d, vs1)

    vs2 = pl.pallas_call(
        _tc2_body,
        out_shape=jax.ShapeDtypeStruct((NPAD, HIDDEN), jnp.float32),
    )(p1, vs1, dis_b, b1r)

    p2 = _msg_pass()(row2d, col2d, ew2d, vs2)

    out = pl.pallas_call(
        _tc3_body,
        out_shape=jax.ShapeDtypeStruct((NPAD, N_CLASSES), jnp.float32),
    )(p2, vs2, dis_b, W2, b2r)

    return out[:N_NODES]


# speedup vs baseline: 28.4596x; 1.0979x over previous
import jax
import jax.numpy as jnp
from jax import lax
from jax.experimental import pallas as pl
from jax.experimental.pallas import tpu as pltpu
from jax.experimental.pallas import tpu_sc as plsc

N_NODES = 10000
N_EDGES = 160000
D_FEAT = 256
HIDDEN = 16
N_CLASSES = 2

NC = 2
NS = 16
NW = NC * NS
CH = 128
NCH = 40
EPW = CH * NCH
RE = N_EDGES // NW
EP = NW * EPW
NPAD = 10240
ZRD = NPAD // NS
ZRM = NPAD // NS
PF = 6
SD = 6

_SC_PARAMS = pltpu.CompilerParams(use_tc_tiling_on_sc=False)


def _zero_tail(ref, zvec):
    for k in range((EPW - RE) // 16):
        ref[pl.ds(RE + 16 * k, 16)] = zvec
    ref[pl.ds(EPW - 16, 16)] = zvec


def _msg_pass():
    mesh = plsc.VectorSubcoreMesh(core_axis_name="c", subcore_axis_name="s")
    scratch = [
        pltpu.VMEM((NCH, CH), jnp.int32),
        pltpu.VMEM((NCH, CH), jnp.int32),
        pltpu.VMEM((NCH, CH), jnp.float32),
        pltpu.VMEM((EPW, HIDDEN), jnp.float32),
        pltpu.VMEM((ZRM, HIDDEN), jnp.float32),
        pltpu.VMEM_SHARED((NPAD, HIDDEN), jnp.float32),
        pltpu.SemaphoreType.DMA,
        pltpu.SemaphoreType.DMA,
    ]

    def body(row_hbm, col_hbm, ew_hbm, table_hbm, out_hbm,
             row_v, col_v, ew_v, msg_v, zbuf, accum, gsem, ssem):
        c = lax.axis_index("c")
        s = lax.axis_index("s")
        wid = c * NS + s

        def zb(i, _):
            zbuf[i] = jnp.zeros((HIDDEN,), jnp.float32)
            return 0
        lax.fori_loop(0, ZRM, zb, 0)
        pltpu.sync_copy(zbuf, accum.at[pl.ds(s * ZRM, ZRM)])

        pltpu.sync_copy(row_hbm.at[pl.ds(wid * NCH, NCH)], row_v)
        pltpu.sync_copy(ew_hbm.at[pl.ds(wid * NCH, NCH)], ew_v)
        pltpu.sync_copy(col_hbm.at[pl.ds(wid * NCH, NCH)], col_v)

        plsc.subcore_barrier()

        def fire_gather(j):
            pltpu.async_copy(table_hbm.at[row_v.at[j]],
                             msg_v.at[pl.ds(j * CH, CH)], gsem)

        def drain_gather(j):
            pltpu.make_async_copy(table_hbm.at[row_v.at[j]],
                                  msg_v.at[pl.ds(j * CH, CH)], gsem).wait()

        def fire_scatter(j):
            pltpu.async_copy(msg_v.at[pl.ds(j * CH, CH)],
                             accum.at[col_v.at[j]], ssem, add=True)

        def drain_scatter(j):
            pltpu.make_async_copy(msg_v.at[pl.ds(j * CH, CH)],
                                  accum.at[col_v.at[j]], ssem).wait()

        for j in range(PF):
            fire_gather(j)

        def step(j, _):
            @pl.when(j + PF < NCH)
            def _():
                fire_gather(j + PF)
            drain_gather(j)
            def sc_mid(k16, _):
                w = ew_v[j, pl.ds(k16 * 16, 16)]
                for l in range(16):
                    k = j * CH + k16 * 16 + l
                    msg_v[k] = msg_v[k] * w[l]
                return 0
            lax.fori_loop(0, CH // 16, sc_mid, 0)
            fire_scatter(j)
            @pl.when(j >= SD)
            def _():
                drain_scatter(j - SD)
            return 0
        lax.fori_loop(0, NCH, step, 0)

        def tail(j, _):
            drain_scatter(j)
            return 0
        lax.fori_loop(NCH - SD, NCH, tail, 0)

        plsc.subcore_barrier()

        pltpu.sync_copy(accum.at[pl.ds(s * ZRM, ZRM)],
                        out_hbm.at[c, pl.ds(s * ZRM, ZRM)])

    return pl.kernel(
        body,
        out_type=jax.ShapeDtypeStruct((NC, NPAD, HIDDEN), jnp.float32),
        mesh=mesh,
        scratch_types=scratch,
        compiler_params=_SC_PARAMS,
    )


def _deg_pass():
    mesh = plsc.VectorSubcoreMesh(core_axis_name="c", subcore_axis_name="s")
    scratch = [
        pltpu.VMEM((NCH, CH), jnp.int32),
        pltpu.VMEM((NCH, CH), jnp.float32),
        pltpu.VMEM((ZRD,), jnp.float32),
        pltpu.VMEM((ZRD, HIDDEN), jnp.float32),
        pltpu.VMEM_SHARED((NPAD,), jnp.float32),
        pltpu.SemaphoreType.DMA,
    ]

    def body(col_hbm, ew_hbm, out_hbm, col_v, ew_v, dbuf, obuf, accum, sem):
        c = lax.axis_index("c")
        s = lax.axis_index("s")
        wid = c * NS + s

        def zb(i, _):
            dbuf[pl.ds(i * 16, 16)] = jnp.zeros((16,), jnp.float32)
            return 0
        lax.fori_loop(0, ZRD // 16, zb, 0)
        pltpu.sync_copy(dbuf, accum.at[pl.ds(s * ZRD, ZRD)])

        pltpu.sync_copy(col_hbm.at[pl.ds(wid * NCH, NCH)], col_v)
        pltpu.sync_copy(ew_hbm.at[pl.ds(wid * NCH, NCH)], ew_v)

        plsc.subcore_barrier()

        def fire(j):
            pltpu.async_copy(ew_v.at[j], accum.at[col_v.at[j]], sem,
                             add=True)

        def drain(j):
            pltpu.make_async_copy(ew_v.at[j], accum.at[col_v.at[j]],
                                  sem).wait()

        for j in range(SD):
            fire(j)

        def step(j, _):
            @pl.when(j + SD < NCH)
            def _():
                fire(j + SD)
            drain(j)
            return 0
        lax.fori_loop(0, NCH, step, 0)

        plsc.subcore_barrier()

        pltpu.sync_copy(accum.at[pl.ds(s * ZRD, ZRD)], dbuf)

        def bc(i, _):
            v = dbuf[pl.ds(i * 16, 16)]
            for l in range(16):
                obuf[i * 16 + l] = jnp.ones((HIDDEN,), jnp.float32) * v[l]
            return 0
        lax.fori_loop(0, ZRD // 16, bc, 0)
        pltpu.sync_copy(obuf, out_hbm.at[c, pl.ds(s * ZRD, ZRD)])

    return pl.kernel(
        body,
        out_type=jax.ShapeDtypeStruct((NC, NPAD, HIDDEN), jnp.float32),
        mesh=mesh,
        scratch_types=scratch,
        compiler_params=_SC_PARAMS,
    )


def _tc_mm_body(x_ref, w1_ref, xw_ref):
    xw_ref[...] = jnp.dot(x_ref[...], w1_ref[...],
                          preferred_element_type=jnp.float32)


def _tc1_body(xw_ref, degp_ref, vs1_ref, dis_ref):
    deg = (degp_ref[0, pl.ds(0, N_NODES), :]
           + degp_ref[1, pl.ds(0, N_NODES), :] + 1.0)
    dis = 1.0 / jnp.sqrt(deg)
    vs1_ref[...] = dis * xw_ref[...]
    dis_ref[...] = dis


def _tc2_body(p1_ref, vs1_ref, dis_ref, b1_ref, vs2_ref):
    dis = dis_ref[...]
    u1 = (p1_ref[0, pl.ds(0, N_NODES), :]
          + p1_ref[1, pl.ds(0, N_NODES), :])
    agg = dis * (u1 + vs1_ref[...]) + b1_ref[...]
    vs2_ref[...] = dis * jnp.maximum(agg, 0.0)


def _tc3_body(p2_ref, vs2_ref, dis_ref, w2_ref, b2_ref, out_ref):
    u2 = (p2_ref[0, pl.ds(0, N_NODES), :]
          + p2_ref[1, pl.ds(0, N_NODES), :])
    t = dis_ref[...] * (u2 + vs2_ref[...])
    logits = jnp.dot(t, w2_ref[...],
                     preferred_element_type=jnp.float32) + b2_ref[...]
    m = jnp.max(logits, axis=1, keepdims=True)
    lse = m + jnp.log(jnp.sum(jnp.exp(logits - m), axis=1, keepdims=True))
    out_ref[...] = logits - lse


def kernel(x, edge_index, edge_attr, W1, b1, W2, b2):
    ei32 = edge_index.astype(jnp.int32)
    ei2d = jnp.pad(ei32, ((0, 0), (0, EP - N_EDGES))).reshape(
        2, NW * NCH, CH)
    row2d = ei2d[0]
    col2d = ei2d[1]
    ew2d = jnp.pad(edge_attr, (0, EP - N_EDGES)).reshape(NW * NCH, CH)
    b1r = b1.reshape(1, HIDDEN)
    b2r = b2.reshape(1, N_CLASSES)

    degp = _deg_pass()(col2d, ew2d)
    xw = pl.pallas_call(
        _tc_mm_body,
        out_shape=jax.ShapeDtypeStruct((N_NODES, HIDDEN), jnp.float32),
    )(x, W1)

    vs1, dis_b = pl.pallas_call(
        _tc1_body,
        out_shape=[jax.ShapeDtypeStruct((N_NODES, HIDDEN), jnp.float32),
                   jax.ShapeDtypeStruct((N_NODES, HIDDEN), jnp.float32)],
    )(xw, degp)

    p1 = _msg_pass()(row2d, col2d, ew2d, vs1)

    vs2 = pl.pallas_call(
        _tc2_body,
        out_shape=jax.ShapeDtypeStruct((N_NODES, HIDDEN), jnp.float32),
    )(p1, vs1, dis_b, b1r)

    p2 = _msg_pass()(row2d, col2d, ew2d, vs2)

    out = pl.pallas_call(
        _tc3_body,
        out_shape=jax.ShapeDtypeStruct((N_NODES, N_CLASSES), jnp.float32),
    )(p2, vs2, dis_b, W2, b2r)

    return out


# speedup vs baseline: 34.7119x; 1.2197x over previous
import jax
import jax.numpy as jnp
from jax import lax
from jax.experimental import pallas as pl
from jax.experimental.pallas import tpu as pltpu
from jax.experimental.pallas import tpu_sc as plsc

N_NODES = 10000
N_EDGES = 160000
D_FEAT = 256
HIDDEN = 16
N_CLASSES = 2

NC = 2
NS = 16
NW = NC * NS
CH = 128
NCH = 40
EPW = CH * NCH
RE = N_EDGES // NW
EP = NW * EPW
NPAD = 10240
ZRD = NPAD // NS
ZRM = NPAD // NS
PF = 6
SD = 6

_SC_PARAMS = pltpu.CompilerParams(use_tc_tiling_on_sc=False)


def _zero_tail(ref, zvec):
    for k in range((EPW - RE) // 16):
        ref[pl.ds(RE + 16 * k, 16)] = zvec
    ref[pl.ds(EPW - 16, 16)] = zvec


def _msg_pass():
    mesh = plsc.VectorSubcoreMesh(core_axis_name="c", subcore_axis_name="s")
    scratch = [
        pltpu.VMEM((NCH, CH), jnp.int32),
        pltpu.VMEM((NCH, CH), jnp.int32),
        pltpu.VMEM((NCH, CH), jnp.float32),
        pltpu.VMEM((EPW, HIDDEN), jnp.float32),
        pltpu.VMEM((ZRM, HIDDEN), jnp.float32),
        pltpu.VMEM_SHARED((NPAD, HIDDEN), jnp.float32),
        pltpu.VMEM_SHARED((NPAD, HIDDEN), jnp.float32),
        pltpu.SemaphoreType.DMA,
        pltpu.SemaphoreType.DMA,
    ]

    def body(row_hbm, col_hbm, ew_hbm, table_hbm, out_hbm,
             row_v, col_v, ew_v, msg_v, zbuf, accum, tbl, gsem, ssem):
        c = lax.axis_index("c")
        s = lax.axis_index("s")
        wid = c * NS + s

        def zb(i, _):
            zbuf[i] = jnp.zeros((HIDDEN,), jnp.float32)
            return 0
        lax.fori_loop(0, ZRM, zb, 0)
        pltpu.sync_copy(zbuf, accum.at[pl.ds(s * ZRM, ZRM)])

        pltpu.sync_copy(table_hbm.at[pl.ds(s * ZRM, ZRM)], zbuf)
        pltpu.sync_copy(zbuf, tbl.at[pl.ds(s * ZRM, ZRM)])

        pltpu.sync_copy(row_hbm.at[pl.ds(wid * NCH, NCH)], row_v)
        pltpu.sync_copy(ew_hbm.at[pl.ds(wid * NCH, NCH)], ew_v)
        pltpu.sync_copy(col_hbm.at[pl.ds(wid * NCH, NCH)], col_v)

        plsc.subcore_barrier()

        def fire_gather(j):
            pltpu.async_copy(tbl.at[row_v.at[j]],
                             msg_v.at[pl.ds(j * CH, CH)], gsem)

        def drain_gather(j):
            pltpu.make_async_copy(tbl.at[row_v.at[j]],
                                  msg_v.at[pl.ds(j * CH, CH)], gsem).wait()

        def fire_scatter(j):
            pltpu.async_copy(msg_v.at[pl.ds(j * CH, CH)],
                             accum.at[col_v.at[j]], ssem, add=True)

        def drain_scatter(j):
            pltpu.make_async_copy(msg_v.at[pl.ds(j * CH, CH)],
                                  accum.at[col_v.at[j]], ssem).wait()

        for j in range(PF):
            fire_gather(j)

        def step(j, _):
            @pl.when(j + PF < NCH)
            def _():
                fire_gather(j + PF)
            drain_gather(j)
            def sc_mid(k16, _):
                w = ew_v[j, pl.ds(k16 * 16, 16)]
                for l in range(16):
                    k = j * CH + k16 * 16 + l
                    msg_v[k] = msg_v[k] * w[l]
                return 0
            lax.fori_loop(0, CH // 16, sc_mid, 0)
            fire_scatter(j)
            @pl.when(j >= SD)
            def _():
                drain_scatter(j - SD)
            return 0
        lax.fori_loop(0, NCH, step, 0)

        def tail(j, _):
            drain_scatter(j)
            return 0
        lax.fori_loop(NCH - SD, NCH, tail, 0)

        plsc.subcore_barrier()

        pltpu.sync_copy(accum.at[pl.ds(s * ZRM, ZRM)],
                        out_hbm.at[c, pl.ds(s * ZRM, ZRM)])

    return pl.kernel(
        body,
        out_type=jax.ShapeDtypeStruct((NC, NPAD, HIDDEN), jnp.float32),
        mesh=mesh,
        scratch_types=scratch,
        compiler_params=_SC_PARAMS,
    )


def _deg_pass():
    mesh = plsc.VectorSubcoreMesh(core_axis_name="c", subcore_axis_name="s")
    scratch = [
        pltpu.VMEM((NCH, CH), jnp.int32),
        pltpu.VMEM((NCH, CH), jnp.float32),
        pltpu.VMEM((ZRD,), jnp.float32),
        pltpu.VMEM((ZRD, HIDDEN), jnp.float32),
        pltpu.VMEM_SHARED((NPAD,), jnp.float32),
        pltpu.SemaphoreType.DMA,
    ]

    def body(col_hbm, ew_hbm, out_hbm, col_v, ew_v, dbuf, obuf, accum, sem):
        c = lax.axis_index("c")
        s = lax.axis_index("s")
        wid = c * NS + s

        def zb(i, _):
            dbuf[pl.ds(i * 16, 16)] = jnp.zeros((16,), jnp.float32)
            return 0
        lax.fori_loop(0, ZRD // 16, zb, 0)
        pltpu.sync_copy(dbuf, accum.at[pl.ds(s * ZRD, ZRD)])

        pltpu.sync_copy(col_hbm.at[pl.ds(wid * NCH, NCH)], col_v)
        pltpu.sync_copy(ew_hbm.at[pl.ds(wid * NCH, NCH)], ew_v)

        plsc.subcore_barrier()

        def fire(j):
            pltpu.async_copy(ew_v.at[j], accum.at[col_v.at[j]], sem,
                             add=True)

        def drain(j):
            pltpu.make_async_copy(ew_v.at[j], accum.at[col_v.at[j]],
                                  sem).wait()

        for j in range(SD):
            fire(j)

        def step(j, _):
            @pl.when(j + SD < NCH)
            def _():
                fire(j + SD)
            drain(j)
            return 0
        lax.fori_loop(0, NCH, step, 0)

        plsc.subcore_barrier()

        pltpu.sync_copy(accum.at[pl.ds(s * ZRD, ZRD)], dbuf)

        def bc(i, _):
            v = dbuf[pl.ds(i * 16, 16)]
            for l in range(16):
                obuf[i * 16 + l] = jnp.ones((HIDDEN,), jnp.float32) * v[l]
            return 0
        lax.fori_loop(0, ZRD // 16, bc, 0)
        pltpu.sync_copy(obuf, out_hbm.at[c, pl.ds(s * ZRD, ZRD)])

    return pl.kernel(
        body,
        out_type=jax.ShapeDtypeStruct((NC, NPAD, HIDDEN), jnp.float32),
        mesh=mesh,
        scratch_types=scratch,
        compiler_params=_SC_PARAMS,
    )


def _tc_mm_body(x_ref, w1_ref, xw_ref):
    xw_ref[...] = jnp.dot(x_ref[...], w1_ref[...],
                          preferred_element_type=jnp.float32)


def _tc1_body(xw_ref, degp_ref, vs1_ref, dis_ref):
    deg = degp_ref[0] + degp_ref[1] + 1.0
    dis = 1.0 / jnp.sqrt(deg)
    vs1_ref[pl.ds(0, N_NODES), :] = dis[:N_NODES] * xw_ref[...]
    vs1_ref[pl.ds(N_NODES, NPAD - N_NODES), :] = jnp.zeros(
        (NPAD - N_NODES, HIDDEN), jnp.float32)
    dis_ref[...] = dis


def _tc2_body(p1_ref, vs1_ref, dis_ref, b1_ref, vs2_ref):
    dis = dis_ref[...]
    u1 = p1_ref[0] + p1_ref[1]
    agg = dis * (u1 + vs1_ref[...]) + b1_ref[...]
    vs2_ref[...] = dis * jnp.maximum(agg, 0.0)


def _tc3_body(p2_ref, vs2_ref, dis_ref, w2_ref, b2_ref, out_ref):
    u2 = (p2_ref[0, pl.ds(0, N_NODES), :]
          + p2_ref[1, pl.ds(0, N_NODES), :])
    t = dis_ref[pl.ds(0, N_NODES), :] * (u2 + vs2_ref[pl.ds(0, N_NODES), :])
    logits = jnp.dot(t, w2_ref[...],
                     preferred_element_type=jnp.float32) + b2_ref[...]
    m = jnp.max(logits, axis=1, keepdims=True)
    lse = m + jnp.log(jnp.sum(jnp.exp(logits - m), axis=1, keepdims=True))
    out_ref[...] = logits - lse


def kernel(x, edge_index, edge_attr, W1, b1, W2, b2):
    ei32 = edge_index.astype(jnp.int32)
    ei2d = jnp.pad(ei32, ((0, 0), (0, EP - N_EDGES))).reshape(
        2, NW * NCH, CH)
    row2d = ei2d[0]
    col2d = ei2d[1]
    ew2d = jnp.pad(edge_attr, (0, EP - N_EDGES)).reshape(NW * NCH, CH)
    b1r = b1.reshape(1, HIDDEN)
    b2r = b2.reshape(1, N_CLASSES)

    degp = _deg_pass()(col2d, ew2d)
    xw = pl.pallas_call(
        _tc_mm_body,
        out_shape=jax.ShapeDtypeStruct((N_NODES, HIDDEN), jnp.float32),
    )(x, W1)

    vs1, dis_b = pl.pallas_call(
        _tc1_body,
        out_shape=[jax.ShapeDtypeStruct((NPAD, HIDDEN), jnp.float32),
                   jax.ShapeDtypeStruct((NPAD, HIDDEN), jnp.float32)],
    )(xw, degp)

    p1 = _msg_pass()(row2d, col2d, ew2d, vs1)

    vs2 = pl.pallas_call(
        _tc2_body,
        out_shape=jax.ShapeDtypeStruct((NPAD, HIDDEN), jnp.float32),
    )(p1, vs1, dis_b, b1r)

    p2 = _msg_pass()(row2d, col2d, ew2d, vs2)

    out = pl.pallas_call(
        _tc3_body,
        out_shape=jax.ShapeDtypeStruct((N_NODES, N_CLASSES), jnp.float32),
    )(p2, vs2, dis_b, W2, b2r)

    return out


# speedup vs baseline: 34.9778x; 1.0077x over previous
import jax
import jax.numpy as jnp
from jax import lax
from jax.experimental import pallas as pl
from jax.experimental.pallas import tpu as pltpu
from jax.experimental.pallas import tpu_sc as plsc

N_NODES = 10000
N_EDGES = 160000
D_FEAT = 256
HIDDEN = 16
N_CLASSES = 2

NC = 2
NS = 16
NW = NC * NS
CH = 128
NCH = 40
EPW = CH * NCH
RE = N_EDGES // NW
EP = NW * EPW
NPAD = 10240
ZRD = NPAD // NS
ZRM = NPAD // NS
PF = 6
SD = 6

_SC_PARAMS = pltpu.CompilerParams(use_tc_tiling_on_sc=False)


def _zero_tail(ref, zvec):
    for k in range((EPW - RE) // 16):
        ref[pl.ds(RE + 16 * k, 16)] = zvec
    ref[pl.ds(EPW - 16, 16)] = zvec


def _msg_pass():
    mesh = plsc.VectorSubcoreMesh(core_axis_name="c", subcore_axis_name="s")
    scratch = [
        pltpu.VMEM((NCH, CH), jnp.int32),
        pltpu.VMEM((NCH, CH), jnp.int32),
        pltpu.VMEM((NCH, CH), jnp.float32),
        pltpu.VMEM((EPW, HIDDEN), jnp.float32),
        pltpu.VMEM((ZRM, HIDDEN), jnp.float32),
        pltpu.VMEM_SHARED((NPAD, HIDDEN), jnp.float32),
        pltpu.VMEM_SHARED((NPAD, HIDDEN), jnp.float32),
        pltpu.SemaphoreType.DMA,
        pltpu.SemaphoreType.DMA,
    ]

    def body(row_hbm, col_hbm, ew_hbm, table_hbm, out_hbm,
             row_v, col_v, ew_v, msg_v, zbuf, accum, tbl, gsem, ssem):
        c = lax.axis_index("c")
        s = lax.axis_index("s")
        wid = c * NS + s

        def zb(i, _):
            zbuf[i] = jnp.zeros((HIDDEN,), jnp.float32)
            return 0
        lax.fori_loop(0, ZRM, zb, 0)
        pltpu.sync_copy(zbuf, accum.at[pl.ds(s * ZRM, ZRM)])

        pltpu.sync_copy(table_hbm.at[pl.ds(s * ZRM, ZRM)], zbuf)
        pltpu.sync_copy(zbuf, tbl.at[pl.ds(s * ZRM, ZRM)])

        pltpu.sync_copy(row_hbm.at[pl.ds(wid * NCH, NCH)], row_v)
        pltpu.sync_copy(ew_hbm.at[pl.ds(wid * NCH, NCH)], ew_v)
        pltpu.sync_copy(col_hbm.at[pl.ds(wid * NCH, NCH)], col_v)

        plsc.subcore_barrier()

        def fire_gather(j):
            pltpu.async_copy(tbl.at[row_v.at[j]],
                             msg_v.at[pl.ds(j * CH, CH)], gsem)

        def drain_gather(j):
            pltpu.make_async_copy(tbl.at[row_v.at[j]],
                                  msg_v.at[pl.ds(j * CH, CH)], gsem).wait()

        def fire_scatter(j):
            pltpu.async_copy(msg_v.at[pl.ds(j * CH, CH)],
                             accum.at[col_v.at[j]], ssem, add=True)

        def drain_scatter(j):
            pltpu.make_async_copy(msg_v.at[pl.ds(j * CH, CH)],
                                  accum.at[col_v.at[j]], ssem).wait()

        for j in range(PF):
            fire_gather(j)

        def step(j, _):
            @pl.when(j + PF < NCH)
            def _():
                fire_gather(j + PF)
            drain_gather(j)
            def sc_mid(k16, _):
                w = ew_v[j, pl.ds(k16 * 16, 16)]
                for l in range(16):
                    k = j * CH + k16 * 16 + l
                    msg_v[k] = msg_v[k] * w[l]
                return 0
            lax.fori_loop(0, CH // 16, sc_mid, 0)
            fire_scatter(j)
            @pl.when(j >= SD)
            def _():
                drain_scatter(j - SD)
            return 0
        lax.fori_loop(0, NCH, step, 0)

        def tail(j, _):
            drain_scatter(j)
            return 0
        lax.fori_loop(NCH - SD, NCH, tail, 0)

        plsc.subcore_barrier()

        pltpu.sync_copy(accum.at[pl.ds(s * ZRM, ZRM)],
                        out_hbm.at[c, pl.ds(s * ZRM, ZRM)])

    return pl.kernel(
        body,
        out_type=jax.ShapeDtypeStruct((NC, NPAD, HIDDEN), jnp.float32),
        mesh=mesh,
        scratch_types=scratch,
        compiler_params=_SC_PARAMS,
    )


RCH = 128
NRCH = ZRM // RCH
NRING = 16


def _msg_pass2():
    mesh = plsc.VectorSubcoreMesh(core_axis_name="c", subcore_axis_name="s")
    scratch = [
        pltpu.VMEM((NCH, CH), jnp.int32),
        pltpu.VMEM((NCH, CH), jnp.int32),
        pltpu.VMEM((NCH, CH), jnp.float32),
        pltpu.VMEM((NRING * CH, HIDDEN), jnp.float32),
        pltpu.VMEM((ZRM, HIDDEN), jnp.float32),
        pltpu.VMEM((RCH, HIDDEN), jnp.float32),
        pltpu.VMEM((RCH, HIDDEN), jnp.float32),
        pltpu.VMEM((RCH, HIDDEN), jnp.float32),
        pltpu.VMEM((RCH, HIDDEN), jnp.float32),
        pltpu.VMEM((RCH, HIDDEN), jnp.float32),
        pltpu.VMEM((HIDDEN,), jnp.float32),
        pltpu.VMEM_SHARED((NPAD, HIDDEN), jnp.float32),
        pltpu.VMEM_SHARED((NPAD, HIDDEN), jnp.float32),
        pltpu.SemaphoreType.DMA,
        pltpu.SemaphoreType.DMA,
    ]

    def body(row_hbm, col_hbm, ew_hbm, p1_hbm, vs1_hbm, dis_hbm, b1_hbm,
             out_hbm, vs2_hbm,
             row_v, col_v, ew_v, msg_v, zbuf, pa, pb, vv, dd, oo, bv,
             accum, tbl, gsem, ssem):
        c = lax.axis_index("c")
        s = lax.axis_index("s")
        wid = c * NS + s

        def zb(i, _):
            zbuf[i] = jnp.zeros((HIDDEN,), jnp.float32)
            return 0
        lax.fori_loop(0, ZRM, zb, 0)
        pltpu.sync_copy(zbuf, accum.at[pl.ds(s * ZRM, ZRM)])

        pltpu.sync_copy(b1_hbm, bv)

        def chunk(k, _):
            r0 = s * ZRM + k * RCH
            pltpu.sync_copy(p1_hbm.at[0, pl.ds(r0, RCH)], pa)
            pltpu.sync_copy(p1_hbm.at[1, pl.ds(r0, RCH)], pb)
            pltpu.sync_copy(vs1_hbm.at[pl.ds(r0, RCH)], vv)
            pltpu.sync_copy(dis_hbm.at[pl.ds(r0, RCH)], dd)
            bvec = bv[...]

            def rowf(i, _):
                d = dd[i]
                agg = d * (pa[i] + pb[i] + vv[i]) + bvec
                oo[i] = d * jnp.maximum(agg, 0.0)
                return 0
            lax.fori_loop(0, RCH, rowf, 0)
            pltpu.sync_copy(oo, tbl.at[pl.ds(r0, RCH)])

            @pl.when(c == 0)
            def _():
                pltpu.sync_copy(oo, vs2_hbm.at[pl.ds(r0, RCH)])
            return 0
        lax.fori_loop(0, NRCH, chunk, 0)

        pltpu.sync_copy(row_hbm.at[pl.ds(wid * NCH, NCH)], row_v)
        pltpu.sync_copy(ew_hbm.at[pl.ds(wid * NCH, NCH)], ew_v)
        pltpu.sync_copy(col_hbm.at[pl.ds(wid * NCH, NCH)], col_v)

        plsc.subcore_barrier()

        def slot(j):
            return (j % NRING) * CH

        def fire_gather(j):
            pltpu.async_copy(tbl.at[row_v.at[j]],
                             msg_v.at[pl.ds(slot(j), CH)], gsem)

        def drain_gather(j):
            pltpu.make_async_copy(tbl.at[row_v.at[j]],
                                  msg_v.at[pl.ds(slot(j), CH)], gsem).wait()

        def fire_scatter(j):
            pltpu.async_copy(msg_v.at[pl.ds(slot(j), CH)],
                             accum.at[col_v.at[j]], ssem, add=True)

        def drain_scatter(j):
            pltpu.make_async_copy(msg_v.at[pl.ds(slot(j), CH)],
                                  accum.at[col_v.at[j]], ssem).wait()

        for j in range(PF):
            fire_gather(j)

        def step(j, _):
            @pl.when(j + PF < NCH)
            def _():
                fire_gather(j + PF)
            drain_gather(j)
            def sc_mid(k16, _):
                w = ew_v[j, pl.ds(k16 * 16, 16)]
                for l in range(16):
                    k = slot(j) + k16 * 16 + l
                    msg_v[k] = msg_v[k] * w[l]
                return 0
            lax.fori_loop(0, CH // 16, sc_mid, 0)
            fire_scatter(j)
            @pl.when(j >= SD)
            def _():
                drain_scatter(j - SD)
            return 0
        lax.fori_loop(0, NCH, step, 0)

        def tail(j, _):
            drain_scatter(j)
            return 0
        lax.fori_loop(NCH - SD, NCH, tail, 0)

        plsc.subcore_barrier()

        pltpu.sync_copy(accum.at[pl.ds(s * ZRM, ZRM)],
                        out_hbm.at[c, pl.ds(s * ZRM, ZRM)])

    return pl.kernel(
        body,
        out_type=[jax.ShapeDtypeStruct((NC, NPAD, HIDDEN), jnp.float32),
                  jax.ShapeDtypeStruct((NPAD, HIDDEN), jnp.float32)],
        mesh=mesh,
        scratch_types=scratch,
        compiler_params=_SC_PARAMS,
    )


def _deg_pass():
    mesh = plsc.VectorSubcoreMesh(core_axis_name="c", subcore_axis_name="s")
    scratch = [
        pltpu.VMEM((NCH, CH), jnp.int32),
        pltpu.VMEM((NCH, CH), jnp.float32),
        pltpu.VMEM((ZRD,), jnp.float32),
        pltpu.VMEM((ZRD, HIDDEN), jnp.float32),
        pltpu.VMEM_SHARED((NPAD,), jnp.float32),
        pltpu.SemaphoreType.DMA,
    ]

    def body(col_hbm, ew_hbm, out_hbm, col_v, ew_v, dbuf, obuf, accum, sem):
        c = lax.axis_index("c")
        s = lax.axis_index("s")
        wid = c * NS + s

        def zb(i, _):
            dbuf[pl.ds(i * 16, 16)] = jnp.zeros((16,), jnp.float32)
            return 0
        lax.fori_loop(0, ZRD // 16, zb, 0)
        pltpu.sync_copy(dbuf, accum.at[pl.ds(s * ZRD, ZRD)])

        pltpu.sync_copy(col_hbm.at[pl.ds(wid * NCH, NCH)], col_v)
        pltpu.sync_copy(ew_hbm.at[pl.ds(wid * NCH, NCH)], ew_v)

        plsc.subcore_barrier()

        def fire(j):
            pltpu.async_copy(ew_v.at[j], accum.at[col_v.at[j]], sem,
                             add=True)

        def drain(j):
            pltpu.make_async_copy(ew_v.at[j], accum.at[col_v.at[j]],
                                  sem).wait()

        for j in range(SD):
            fire(j)

        def step(j, _):
            @pl.when(j + SD < NCH)
            def _():
                fire(j + SD)
            drain(j)
            return 0
        lax.fori_loop(0, NCH, step, 0)

        plsc.subcore_barrier()

        pltpu.sync_copy(accum.at[pl.ds(s * ZRD, ZRD)], dbuf)

        def bc(i, _):
            v = dbuf[pl.ds(i * 16, 16)]
            for l in range(16):
                obuf[i * 16 + l] = jnp.ones((HIDDEN,), jnp.float32) * v[l]
            return 0
        lax.fori_loop(0, ZRD // 16, bc, 0)
        pltpu.sync_copy(obuf, out_hbm.at[c, pl.ds(s * ZRD, ZRD)])

    return pl.kernel(
        body,
        out_type=jax.ShapeDtypeStruct((NC, NPAD, HIDDEN), jnp.float32),
        mesh=mesh,
        scratch_types=scratch,
        compiler_params=_SC_PARAMS,
    )


def _tc_mm_body(x_ref, w1_ref, xw_ref):
    xw_ref[...] = jnp.dot(x_ref[...], w1_ref[...],
                          preferred_element_type=jnp.float32)


def _tc1_body(xw_ref, degp_ref, vs1_ref, dis_ref):
    deg = degp_ref[0] + degp_ref[1] + 1.0
    dis = 1.0 / jnp.sqrt(deg)
    vs1_ref[pl.ds(0, N_NODES), :] = dis[:N_NODES] * xw_ref[...]
    vs1_ref[pl.ds(N_NODES, NPAD - N_NODES), :] = jnp.zeros(
        (NPAD - N_NODES, HIDDEN), jnp.float32)
    dis_ref[...] = dis


def _tc3_body(p2_ref, vs2_ref, dis_ref, w2_ref, b2_ref, out_ref):
    u2 = (p2_ref[0, pl.ds(0, N_NODES), :]
          + p2_ref[1, pl.ds(0, N_NODES), :])
    t = dis_ref[pl.ds(0, N_NODES), :] * (u2 + vs2_ref[pl.ds(0, N_NODES), :])
    logits = jnp.dot(t, w2_ref[...],
                     preferred_element_type=jnp.float32) + b2_ref[...]
    m = jnp.max(logits, axis=1, keepdims=True)
    lse = m + jnp.log(jnp.sum(jnp.exp(logits - m), axis=1, keepdims=True))
    out_ref[...] = logits - lse


def kernel(x, edge_index, edge_attr, W1, b1, W2, b2):
    ei32 = edge_index.astype(jnp.int32)
    ei2d = jnp.pad(ei32, ((0, 0), (0, EP - N_EDGES))).reshape(
        2, NW * NCH, CH)
    row2d = ei2d[0]
    col2d = ei2d[1]
    ew2d = jnp.pad(edge_attr, (0, EP - N_EDGES)).reshape(NW * NCH, CH)
    b2r = b2.reshape(1, N_CLASSES)

    degp = _deg_pass()(col2d, ew2d)
    xw = pl.pallas_call(
        _tc_mm_body,
        out_shape=jax.ShapeDtypeStruct((N_NODES, HIDDEN), jnp.float32),
    )(x, W1)

    vs1, dis_b = pl.pallas_call(
        _tc1_body,
        out_shape=[jax.ShapeDtypeStruct((NPAD, HIDDEN), jnp.float32),
                   jax.ShapeDtypeStruct((NPAD, HIDDEN), jnp.float32)],
    )(xw, degp)

    p1 = _msg_pass()(row2d, col2d, ew2d, vs1)

    p2, vs2 = _msg_pass2()(row2d, col2d, ew2d, p1, vs1, dis_b, b1)

    out = pl.pallas_call(
        _tc3_body,
        out_shape=jax.ShapeDtypeStruct((N_NODES, N_CLASSES), jnp.float32),
    )(p2, vs2, dis_b, W2, b2r)

    return out
